# baseline (speedup 1.0000x reference)
import functools

import jax
import jax.numpy as jnp
from jax import lax
from jax.experimental import pallas as pl
from jax.experimental.pallas import tpu as pltpu
from jax.experimental.pallas import tpu_sc as plsc

N, P, CP, C0, K = 16, 2048, 3, 64, 16
NP = N * P
NPK = N * P * K
EPS = 1e-3
SLOPE = 0.1

ROWS_B = 4096
TPF = 1024
TRK = 1024


def _leaky(x):
    return jnp.where(x > 0, x, SLOPE * x)


def _knn_body(pts_ref, ptsT_ref, f_ref, wab_ref, wb_ref, wsc_ref,
              idx_ref, fa_ref, fb_ref, scp_ref, scstats_ref, acc):
    n = pl.program_id(0)
    j = pl.program_id(1)

    @pl.when((n == 0) & (j == 0))
    def _():
        acc[...] = jnp.zeros_like(acc)

    p = pts_ref[0]
    pT = ptsT_ref[0]
    g = jnp.dot(p.astype(jnp.bfloat16), pT.astype(jnp.bfloat16),
                preferred_element_type=jnp.float32)
    r_col = jnp.sum(p * p, axis=1, keepdims=True)
    r_row = jnp.sum(pT * pT, axis=0, keepdims=True)
    d = r_col - 2.0 * g + r_row

    col = jax.lax.broadcasted_iota(jnp.int32, (TRK, P), 1)
    inf = jnp.float32(jnp.inf)

    cols = []
    for kk in range(K + 1):
        mn = jnp.min(d, axis=1, keepdims=True)
        cand = jnp.where(d == mn, col, jnp.int32(P))
        ii = jnp.min(cand, axis=1, keepdims=True)
        if kk > 0:
            cols.append(ii)
        d = jnp.where(cand == ii, inf, d)
    idx_ref[...] = jnp.concatenate(cols, axis=1) + n * P

    f = f_ref[...]
    fa_ref[...] = jnp.dot(f, wab_ref[...], preferred_element_type=jnp.float32)
    fb_ref[...] = jnp.dot(f, wb_ref[...], preferred_element_type=jnp.float32)
    scp = jnp.dot(f, wsc_ref[...], preferred_element_type=jnp.float32)
    scp_ref[...] = scp
    acc[0:1, :] += jnp.sum(scp, axis=0, keepdims=True)
    acc[1:2, :] += jnp.sum(scp * scp, axis=0, keepdims=True)
    scstats_ref[...] = acc[...]


def _knn(ptsp, ptsT, f2, wab, wb, wsc):
    PJ = P // TRK
    return pl.pallas_call(
        _knn_body,
        grid=(N, PJ),
        in_specs=[
            pl.BlockSpec((1, TRK, 8), lambda n, j: (n, j, 0)),
            pl.BlockSpec((1, 8, P), lambda n, j: (n, 0, 0)),
            pl.BlockSpec((TRK, C0), lambda n, j: (n * PJ + j, 0)),
            pl.BlockSpec((C0, C0), lambda n, j: (0, 0)),
            pl.BlockSpec((C0, C0), lambda n, j: (0, 0)),
            pl.BlockSpec((C0, C0), lambda n, j: (0, 0)),
        ],
        out_specs=[
            pl.BlockSpec((TRK, K), lambda n, j: (n * PJ + j, 0)),
            pl.BlockSpec((TRK, C0), lambda n, j: (n * PJ + j, 0)),
            pl.BlockSpec((TRK, C0), lambda n, j: (n * PJ + j, 0)),
            pl.BlockSpec((TRK, C0), lambda n, j: (n * PJ + j, 0)),
            pl.BlockSpec((2, C0), lambda n, j: (0, 0)),
        ],
        out_shape=[
            jax.ShapeDtypeStruct((NP, K), jnp.int32),
            jax.ShapeDtypeStruct((NP, C0), jnp.float32),
            jax.ShapeDtypeStruct((NP, C0), jnp.float32),
            jax.ShapeDtypeStruct((NP, C0), jnp.float32),
            jax.ShapeDtypeStruct((2, C0), jnp.float32),
        ],
        scratch_shapes=[pltpu.VMEM((2, C0), jnp.float32)],
    )(ptsp, ptsT, f2, wab, wb, wsc)


def _stats_body_nl(nl):
    def body(h_ref, cs_ref, ws_ref, stats_ref, acc):
        i = pl.program_id(0)

        @pl.when(i == 0)
        def _():
            acc[...] = jnp.zeros_like(acc)

        x = h_ref[...]
        for l in range(nl):
            x = _leaky(x * cs_ref[l, 0:1, :] + cs_ref[l, 1:2, :])
            x = jnp.dot(x, ws_ref[l], preferred_element_type=jnp.float32)
        acc[0:1, :] += jnp.sum(x, axis=0, keepdims=True)
        acc[1:2, :] += jnp.sum(x * x, axis=0, keepdims=True)
        stats_ref[...] = acc[...]
    return body


def _stats_chain(h, cs, ws):
    nl = ws.shape[0]
    rb = ROWS_B // 2
    return pl.pallas_call(
        _stats_body_nl(nl),
        grid=(NR2 // rb,),
        in_specs=[
            pl.BlockSpec((rb, 2 * C0), lambda i: (i, 0)),
            pl.BlockSpec((nl, 2, 2 * C0), lambda i: (0, 0, 0)),
            pl.BlockSpec((nl, 2 * C0, 2 * C0), lambda i: (0, 0, 0)),
        ],
        out_specs=pl.BlockSpec((2, 2 * C0), lambda i: (0, 0)),
        out_shape=jax.ShapeDtypeStruct((2, 2 * C0), jnp.float32),
        scratch_shapes=[pltpu.VMEM((2, 2 * C0), jnp.float32)],
    )(h, cs, ws)


def _final_body(h_ref, cs_ref, ws_ref, sc_ref, csc_ref, out_ref):
    x = h_ref[...]
    for l in range(2):
        x = _leaky(x * cs_ref[l, 0:1, :] + cs_ref[l, 1:2, :])
        x = jnp.dot(x, ws_ref[l], preferred_element_type=jnp.float32)
    x = _leaky(x * cs_ref[2, 0:1, :] + cs_ref[2, 1:2, :])
    pooled = jnp.mean(x.reshape(TPF, K // 2, 2 * C0), axis=1)
    pooled = 0.5 * (pooled[:, :C0] + pooled[:, C0:])
    sc = sc_ref[...] * csc_ref[0:1, :] + csc_ref[1:2, :]
    out_ref[...] = _leaky(pooled + sc)


def _final(h1, cs, ws, sc_pre, csc):
    rpb = TPF * (K // 2)
    return pl.pallas_call(
        _final_body,
        grid=(NP // TPF,),
        in_specs=[
            pl.BlockSpec((rpb, 2 * C0), lambda i: (i, 0)),
            pl.BlockSpec((3, 2, 2 * C0), lambda i: (0, 0, 0)),
            pl.BlockSpec((2, 2 * C0, 2 * C0), lambda i: (0, 0, 0)),
            pl.BlockSpec((TPF, C0), lambda i: (i, 0)),
            pl.BlockSpec((2, C0), lambda i: (0, 0)),
        ],
        out_specs=pl.BlockSpec((TPF, C0), lambda i: (i, 0)),
        out_shape=jax.ShapeDtypeStruct((NP, C0), jnp.float32),
    )(h1, cs, ws, sc_pre, csc)


NW = 32
PTS_W = NP // NW
PTS_C = 32
NCH = PTS_W // PTS_C
RPC = PTS_C * K
NR2 = NPK // 2


def _gather_body(fa_hbm, fb_hbm, idx_hbm, h1_hbm, stats_hbm,
                 idx_v0, idx_v1, fa_v0, fa_v1, g_v0, g_v1, o_v, acc_v,
                 sem_l, sem_g0, sem_g1):
    wid = lax.axis_index("s") * 2 + lax.axis_index("c")
    idx_vs, fa_vs, g_vs = (idx_v0, idx_v1), (fa_v0, fa_v1), (g_v0, g_v1)
    sem_gs = (sem_g0, sem_g1)

    def pair_fn(i, accs):
        bases = []
        loads = []
        for b in (0, 1):
            c = 2 * i + b
            base_pt = pl.multiple_of(wid * PTS_W + c * PTS_C, PTS_C)
            base_row = pl.multiple_of(base_pt * K, RPC)
            bases.append((base_pt, base_row))
            loads.append(pltpu.async_copy(
                idx_hbm.at[pl.ds(pl.multiple_of(base_row // 128, RPC // 128),
                                 RPC // 128)], idx_vs[b], sem_l))
            loads.append(pltpu.async_copy(
                fa_hbm.at[pl.ds(base_pt, PTS_C)], fa_vs[b], sem_l))
        for cp in loads:
            cp.wait()
        gathers = []
        for b in (0, 1):
            gathers.append([pltpu.async_copy(fb_hbm.at[idx_vs[b].at[j]],
                                             g_vs[b].at[pl.ds(j * 128, 128)],
                                             sem_gs[b])
                            for j in range(RPC // 128)])

        for b in (0, 1):
            for cp in gathers[b]:
                cp.wait()
            g_v = g_vs[b]
            fa_v = fa_vs[b]

            def pt_fn(p, acc2):
                a = list(acc2)
                rb = p * K
                ob = p * (K // 2)
                for kk in range(K):
                    for s in range(4):
                        h = (g_v[rb + kk, pl.ds(s * 16, 16)]
                             + fa_v[p, pl.ds(s * 16, 16)])
                        o_v[ob + kk // 2, pl.ds((kk % 2) * C0 + s * 16, 16)] = h
                        a[s] = a[s] + h
                        a[4 + s] = a[4 + s] + h * h
                return tuple(a)

            accs = lax.fori_loop(0, PTS_C, pt_fn, accs)
            base_row = bases[b][1]
            pltpu.sync_copy(o_v, h1_hbm.at[pl.ds(
                pl.multiple_of(base_row // 2, RPC // 2), RPC // 2)])
        return accs

    accs = lax.fori_loop(0, NCH // 2, pair_fn,
                         tuple([jnp.zeros((16,), jnp.float32)] * 8))
    for s in range(4):
        acc_v[0, pl.ds(s * 16, 16)] = accs[s]
        acc_v[1, pl.ds(s * 16, 16)] = accs[4 + s]
    pltpu.sync_copy(acc_v, stats_hbm.at[wid])


def _sc_gather(Fa, Fb, idx2):
    return pl.kernel(
        _gather_body,
        out_type=[
            jax.ShapeDtypeStruct((NR2, 2 * C0), jnp.float32),
            jax.ShapeDtypeStruct((NW, 2, C0), jnp.float32),
        ],
        mesh=plsc.VectorSubcoreMesh(core_axis_name="c", subcore_axis_name="s"),
        compiler_params=pltpu.CompilerParams(use_tc_tiling_on_sc=False),
        scratch_types=[
            pltpu.VMEM((RPC // 128, 128), jnp.int32),
            pltpu.VMEM((RPC // 128, 128), jnp.int32),
            pltpu.VMEM((PTS_C, C0), jnp.float32),
            pltpu.VMEM((PTS_C, C0), jnp.float32),
            pltpu.VMEM((RPC, C0), jnp.float32),
            pltpu.VMEM((RPC, C0), jnp.float32),
            pltpu.VMEM((RPC // 2, 2 * C0), jnp.float32),
            pltpu.VMEM((2, C0), jnp.float32),
            pltpu.SemaphoreType.DMA,
            pltpu.SemaphoreType.DMA,
            pltpu.SemaphoreType.DMA,
        ],
    )(Fa, Fb, idx2)


def _coef(stats, cnt, g, b):
    m = stats[0] / cnt
    v = stats[1] / cnt - m * m
    s = g * jax.lax.rsqrt(v + EPS)
    return jnp.stack([s, b - m * s])


def _dup(c):
    return jnp.concatenate([c, c], axis=1)


def _fold(st):
    return st[:, :C0] + st[:, C0:]


def _bd(w):
    z = jnp.zeros((C0, C0), w.dtype)
    return jnp.block([[w, z], [z, w]])


def kernel(points, features, W0, W1, W2, Wsc, g0, b0, g1, b1, g2, b2, gsc, bsc):
    f2 = features.reshape(NP, C0)
    W0a, W0b = W0[:C0], W0[C0:]
    ptsp = jnp.pad(points, ((0, 0), (0, 0), (0, 8 - CP)))
    ptsT = ptsp.transpose(0, 2, 1)

    idxg, Fa, Fb, sc_pre, stats_sc = _knn(ptsp, ptsT, f2, W0a - W0b, W0b, Wsc)

    h1, statsp = _sc_gather(Fa, Fb, idxg.reshape(NPK // 128, 128))
    stats1 = jnp.sum(statsp, axis=0)

    W1d, W2d = _bd(W1), _bd(W2)
    c1 = _coef(stats1, NPK, g0, b0)
    stats2 = _stats_chain(h1, _dup(c1)[None], W1d[None])
    c2 = _coef(_fold(stats2), NPK, g1, b1)
    stats3 = _stats_chain(h1, jnp.stack([_dup(c1), _dup(c2)]),
                          jnp.stack([W1d, W2d]))
    c3 = _coef(_fold(stats3), NPK, g2, b2)

    csc = _coef(stats_sc, NP, gsc, bsc)

    out = _final(h1, jnp.stack([_dup(c1), _dup(c2), _dup(c3)]),
                 jnp.stack([W1d, W2d]), sc_pre, csc)
    return out.reshape(N, P, C0)


# speedup vs baseline: 1.2004x; 1.2004x over previous
import functools

import jax
import jax.numpy as jnp
from jax import lax
from jax.experimental import pallas as pl
from jax.experimental.pallas import tpu as pltpu
from jax.experimental.pallas import tpu_sc as plsc

N, P, CP, C0, K = 16, 2048, 3, 64, 16
NP = N * P
NPK = N * P * K
EPS = 1e-3
SLOPE = 0.1

ROWS_B = 8192
TPF = 2048
TRK = 512


def _leaky(x):
    return jnp.where(x > 0, x, SLOPE * x)


def _knn_body(pts_ref, ptsT_ref, f_ref, wab_ref, wb_ref, wsc_ref,
              idx_ref, fa_ref, fb_ref, scp_ref, scstats_ref, acc):
    n = pl.program_id(0)
    j = pl.program_id(1)

    @pl.when((n == 0) & (j == 0))
    def _():
        acc[...] = jnp.zeros_like(acc)

    p = pts_ref[0]
    pT = ptsT_ref[0]
    g = jnp.dot(p.astype(jnp.bfloat16), pT.astype(jnp.bfloat16),
                preferred_element_type=jnp.float32)
    r_col = jnp.sum(p * p, axis=1, keepdims=True)
    r_row = jnp.sum(pT * pT, axis=0, keepdims=True)
    d = r_col - 2.0 * g + r_row

    col = jax.lax.broadcasted_iota(jnp.int32, (TRK, P), 1)
    inf = jnp.float32(jnp.inf)

    cols = []
    for kk in range(K + 1):
        mn = jnp.min(d, axis=1, keepdims=True)
        cand = jnp.where(d == mn, col, jnp.int32(P))
        ii = jnp.min(cand, axis=1, keepdims=True)
        if kk > 0:
            cols.append(ii)
        d = jnp.where(cand == ii, inf, d)
    idx_ref[...] = jnp.concatenate(cols, axis=1) + n * P

    f = f_ref[...]
    fa_ref[...] = jnp.dot(f, wab_ref[...], preferred_element_type=jnp.float32)
    fb_ref[...] = jnp.dot(f, wb_ref[...], preferred_element_type=jnp.float32)
    scp = jnp.dot(f, wsc_ref[...], preferred_element_type=jnp.float32)
    scp_ref[...] = scp
    acc[0:1, :] += jnp.sum(scp, axis=0, keepdims=True)
    acc[1:2, :] += jnp.sum(scp * scp, axis=0, keepdims=True)
    scstats_ref[...] = acc[...]


def _knn(ptsp, ptsT, f2, wab, wb, wsc):
    PJ = P // TRK
    return pl.pallas_call(
        _knn_body,
        grid=(N, PJ),
        in_specs=[
            pl.BlockSpec((1, TRK, 8), lambda n, j: (n, j, 0)),
            pl.BlockSpec((1, 8, P), lambda n, j: (n, 0, 0)),
            pl.BlockSpec((TRK, C0), lambda n, j: (n * PJ + j, 0)),
            pl.BlockSpec((C0, C0), lambda n, j: (0, 0)),
            pl.BlockSpec((C0, C0), lambda n, j: (0, 0)),
            pl.BlockSpec((C0, C0), lambda n, j: (0, 0)),
        ],
        out_specs=[
            pl.BlockSpec((TRK, K), lambda n, j: (n * PJ + j, 0)),
            pl.BlockSpec((TRK, C0), lambda n, j: (n * PJ + j, 0)),
            pl.BlockSpec((TRK, C0), lambda n, j: (n * PJ + j, 0)),
            pl.BlockSpec((TRK, C0), lambda n, j: (n * PJ + j, 0)),
            pl.BlockSpec((2, C0), lambda n, j: (0, 0)),
        ],
        out_shape=[
            jax.ShapeDtypeStruct((NP, K), jnp.int32),
            jax.ShapeDtypeStruct((NP, C0), jnp.float32),
            jax.ShapeDtypeStruct((NP, C0), jnp.float32),
            jax.ShapeDtypeStruct((NP, C0), jnp.float32),
            jax.ShapeDtypeStruct((2, C0), jnp.float32),
        ],
        scratch_shapes=[pltpu.VMEM((2, C0), jnp.float32)],
    )(ptsp, ptsT, f2, wab, wb, wsc)


def _stats_body_nl(nl):
    def body(h_ref, cs_ref, ws_ref, stats_ref, acc):
        i = pl.program_id(0)

        @pl.when(i == 0)
        def _():
            acc[...] = jnp.zeros_like(acc)

        x = h_ref[...]
        for l in range(nl):
            x = _leaky(x * cs_ref[l, 0:1, :] + cs_ref[l, 1:2, :])
            x = jnp.dot(x, ws_ref[l], preferred_element_type=jnp.float32)
        acc[0:1, :] += jnp.sum(x, axis=0, keepdims=True)
        acc[1:2, :] += jnp.sum(x * x, axis=0, keepdims=True)
        stats_ref[...] = acc[...]
    return body


def _stats_chain(h, cs, ws):
    nl = ws.shape[0]
    rb = ROWS_B // 2
    return pl.pallas_call(
        _stats_body_nl(nl),
        grid=(NR2 // rb,),
        in_specs=[
            pl.BlockSpec((rb, 2 * C0), lambda i: (i, 0)),
            pl.BlockSpec((nl, 2, 2 * C0), lambda i: (0, 0, 0)),
            pl.BlockSpec((nl, 2 * C0, 2 * C0), lambda i: (0, 0, 0)),
        ],
        out_specs=pl.BlockSpec((2, 2 * C0), lambda i: (0, 0)),
        out_shape=jax.ShapeDtypeStruct((2, 2 * C0), jnp.float32),
        scratch_shapes=[pltpu.VMEM((2, 2 * C0), jnp.float32)],
    )(h, cs, ws)


def _final_body(h_ref, cs_ref, ws_ref, sc_ref, csc_ref, out_ref):
    x = h_ref[...]
    for l in range(2):
        x = _leaky(x * cs_ref[l, 0:1, :] + cs_ref[l, 1:2, :])
        x = jnp.dot(x, ws_ref[l], preferred_element_type=jnp.float32)
    x = _leaky(x * cs_ref[2, 0:1, :] + cs_ref[2, 1:2, :])
    pooled = jnp.mean(x.reshape(TPF, K // 2, 2 * C0), axis=1)
    pooled = 0.5 * (pooled[:, :C0] + pooled[:, C0:])
    sc = sc_ref[...] * csc_ref[0:1, :] + csc_ref[1:2, :]
    out_ref[...] = _leaky(pooled + sc)


def _final(h1, cs, ws, sc_pre, csc):
    rpb = TPF * (K // 2)
    return pl.pallas_call(
        _final_body,
        grid=(NP // TPF,),
        in_specs=[
            pl.BlockSpec((rpb, 2 * C0), lambda i: (i, 0)),
            pl.BlockSpec((3, 2, 2 * C0), lambda i: (0, 0, 0)),
            pl.BlockSpec((2, 2 * C0, 2 * C0), lambda i: (0, 0, 0)),
            pl.BlockSpec((TPF, C0), lambda i: (i, 0)),
            pl.BlockSpec((2, C0), lambda i: (0, 0)),
        ],
        out_specs=pl.BlockSpec((TPF, C0), lambda i: (i, 0)),
        out_shape=jax.ShapeDtypeStruct((NP, C0), jnp.float32),
    )(h1, cs, ws, sc_pre, csc)


NW = 32
PTS_W = NP // NW
PTS_C = 32
NCH = PTS_W // PTS_C
RPC = PTS_C * K
NR2 = NPK // 2


def _gather_body(fa_hbm, fb_hbm, idx_hbm, h1_hbm, stats_hbm,
                 idx_v0, idx_v1, fa_v0, fa_v1, g_v0, g_v1, o_v, acc_v,
                 sem_l, sem_g0, sem_g1):
    wid = lax.axis_index("s") * 2 + lax.axis_index("c")
    idx_vs, fa_vs, g_vs = (idx_v0, idx_v1), (fa_v0, fa_v1), (g_v0, g_v1)
    sem_gs = (sem_g0, sem_g1)

    def pair_fn(i, accs):
        bases = []
        loads = []
        for b in (0, 1):
            c = 2 * i + b
            base_pt = pl.multiple_of(wid * PTS_W + c * PTS_C, PTS_C)
            base_row = pl.multiple_of(base_pt * K, RPC)
            bases.append((base_pt, base_row))
            loads.append(pltpu.async_copy(
                idx_hbm.at[pl.ds(pl.multiple_of(base_row // 128, RPC // 128),
                                 RPC // 128)], idx_vs[b], sem_l))
            loads.append(pltpu.async_copy(
                fa_hbm.at[pl.ds(base_pt, PTS_C)], fa_vs[b], sem_l))
        for cp in loads:
            cp.wait()
        gathers = []
        for b in (0, 1):
            gathers.append([pltpu.async_copy(fb_hbm.at[idx_vs[b].at[j]],
                                             g_vs[b].at[pl.ds(j * 128, 128)],
                                             sem_gs[b])
                            for j in range(RPC // 128)])

        for b in (0, 1):
            for cp in gathers[b]:
                cp.wait()
            g_v = g_vs[b]
            fa_v = fa_vs[b]

            def pt_fn(p, acc2):
                a = list(acc2)
                rb = p * K
                ob = p * (K // 2)
                for kk in range(K):
                    for s in range(4):
                        h = (g_v[rb + kk, pl.ds(s * 16, 16)]
                             + fa_v[p, pl.ds(s * 16, 16)])
                        o_v[ob + kk // 2, pl.ds((kk % 2) * C0 + s * 16, 16)] = h
                        a[s] = a[s] + h
                        a[4 + s] = a[4 + s] + h * h
                return tuple(a)

            accs = lax.fori_loop(0, PTS_C, pt_fn, accs)
            base_row = bases[b][1]
            pltpu.sync_copy(o_v, h1_hbm.at[pl.ds(
                pl.multiple_of(base_row // 2, RPC // 2), RPC // 2)])
        return accs

    accs = lax.fori_loop(0, NCH // 2, pair_fn,
                         tuple([jnp.zeros((16,), jnp.float32)] * 8))
    for s in range(4):
        acc_v[0, pl.ds(s * 16, 16)] = accs[s]
        acc_v[1, pl.ds(s * 16, 16)] = accs[4 + s]
    pltpu.sync_copy(acc_v, stats_hbm.at[wid])


def _sc_gather(Fa, Fb, idx2):
    return pl.kernel(
        _gather_body,
        out_type=[
            jax.ShapeDtypeStruct((NR2, 2 * C0), jnp.float32),
            jax.ShapeDtypeStruct((NW, 2, C0), jnp.float32),
        ],
        mesh=plsc.VectorSubcoreMesh(core_axis_name="c", subcore_axis_name="s"),
        compiler_params=pltpu.CompilerParams(use_tc_tiling_on_sc=False),
        scratch_types=[
            pltpu.VMEM((RPC // 128, 128), jnp.int32),
            pltpu.VMEM((RPC // 128, 128), jnp.int32),
            pltpu.VMEM((PTS_C, C0), jnp.float32),
            pltpu.VMEM((PTS_C, C0), jnp.float32),
            pltpu.VMEM((RPC, C0), jnp.float32),
            pltpu.VMEM((RPC, C0), jnp.float32),
            pltpu.VMEM((RPC // 2, 2 * C0), jnp.float32),
            pltpu.VMEM((2, C0), jnp.float32),
            pltpu.SemaphoreType.DMA,
            pltpu.SemaphoreType.DMA,
            pltpu.SemaphoreType.DMA,
        ],
    )(Fa, Fb, idx2)


def _coef(stats, cnt, g, b):
    m = stats[0] / cnt
    v = stats[1] / cnt - m * m
    s = g * jax.lax.rsqrt(v + EPS)
    return jnp.stack([s, b - m * s])


def _dup(c):
    return jnp.concatenate([c, c], axis=1)


def _fold(st):
    return st[:, :C0] + st[:, C0:]


def _bd(w):
    z = jnp.zeros((C0, C0), w.dtype)
    return jnp.block([[w, z], [z, w]])


def kernel(points, features, W0, W1, W2, Wsc, g0, b0, g1, b1, g2, b2, gsc, bsc):
    f2 = features.reshape(NP, C0)
    W0a, W0b = W0[:C0], W0[C0:]
    ptsp = jnp.pad(points, ((0, 0), (0, 0), (0, 8 - CP)))
    ptsT = ptsp.transpose(0, 2, 1)

    idxg, Fa, Fb, sc_pre, stats_sc = _knn(ptsp, ptsT, f2, W0a - W0b, W0b, Wsc)

    h1, statsp = _sc_gather(Fa, Fb, idxg.reshape(NPK // 128, 128))
    stats1 = jnp.sum(statsp, axis=0)

    W1d, W2d = _bd(W1), _bd(W2)
    c1 = _coef(stats1, NPK, g0, b0)
    stats2 = _stats_chain(h1, _dup(c1)[None], W1d[None])
    c2 = _coef(_fold(stats2), NPK, g1, b1)
    stats3 = _stats_chain(h1, jnp.stack([_dup(c1), _dup(c2)]),
                          jnp.stack([W1d, W2d]))
    c3 = _coef(_fold(stats3), NPK, g2, b2)

    csc = _coef(stats_sc, NP, gsc, bsc)

    out = _final(h1, jnp.stack([_dup(c1), _dup(c2), _dup(c3)]),
                 jnp.stack([W1d, W2d]), sc_pre, csc)
    return out.reshape(N, P, C0)


# speedup vs baseline: 1.2228x; 1.0186x over previous
import functools

import jax
import jax.numpy as jnp
from jax import lax
from jax.experimental import pallas as pl
from jax.experimental.pallas import tpu as pltpu
from jax.experimental.pallas import tpu_sc as plsc

N, P, CP, C0, K = 16, 2048, 3, 64, 16
NP = N * P
NPK = N * P * K
EPS = 1e-3
SLOPE = 0.1

ROWS_B = 16384
TPF = 2048
TRK = 512


def _leaky(x):
    return jnp.where(x > 0, x, SLOPE * x)


def _knn_body(pts_ref, ptsT_ref, f_ref, wab_ref, wb_ref, wsc_ref,
              idx_ref, fa_ref, fb_ref, scp_ref, scstats_ref, acc):
    n = pl.program_id(0)
    j = pl.program_id(1)

    @pl.when((n == 0) & (j == 0))
    def _():
        acc[...] = jnp.zeros_like(acc)

    p = pts_ref[0]
    pT = ptsT_ref[0]
    g = jnp.dot(p.astype(jnp.bfloat16), pT.astype(jnp.bfloat16),
                preferred_element_type=jnp.float32)
    r_col = jnp.sum(p * p, axis=1, keepdims=True)
    r_row = jnp.sum(pT * pT, axis=0, keepdims=True)
    d = r_col - 2.0 * g + r_row

    col = jax.lax.broadcasted_iota(jnp.int32, (TRK, P), 1)
    inf = jnp.float32(jnp.inf)

    cols = []
    for kk in range(K + 1):
        mn = jnp.min(d, axis=1, keepdims=True)
        cand = jnp.where(d == mn, col, jnp.int32(P))
        ii = jnp.min(cand, axis=1, keepdims=True)
        if kk > 0:
            cols.append(ii)
        d = jnp.where(cand == ii, inf, d)
    idx_ref[...] = jnp.concatenate(cols, axis=1) + n * P

    f = f_ref[...]
    fa_ref[...] = jnp.dot(f, wab_ref[...], preferred_element_type=jnp.float32)
    fb_ref[...] = jnp.dot(f, wb_ref[...], preferred_element_type=jnp.float32)
    scp = jnp.dot(f, wsc_ref[...], preferred_element_type=jnp.float32)
    scp_ref[...] = scp
    acc[0:1, :] += jnp.sum(scp, axis=0, keepdims=True)
    acc[1:2, :] += jnp.sum(scp * scp, axis=0, keepdims=True)
    scstats_ref[...] = acc[...]


def _knn(ptsp, ptsT, f2, wab, wb, wsc):
    PJ = P // TRK
    return pl.pallas_call(
        _knn_body,
        grid=(N, PJ),
        in_specs=[
            pl.BlockSpec((1, TRK, 8), lambda n, j: (n, j, 0)),
            pl.BlockSpec((1, 8, P), lambda n, j: (n, 0, 0)),
            pl.BlockSpec((TRK, C0), lambda n, j: (n * PJ + j, 0)),
            pl.BlockSpec((C0, C0), lambda n, j: (0, 0)),
            pl.BlockSpec((C0, C0), lambda n, j: (0, 0)),
            pl.BlockSpec((C0, C0), lambda n, j: (0, 0)),
        ],
        out_specs=[
            pl.BlockSpec((TRK, K), lambda n, j: (n * PJ + j, 0)),
            pl.BlockSpec((TRK, C0), lambda n, j: (n * PJ + j, 0)),
            pl.BlockSpec((TRK, C0), lambda n, j: (n * PJ + j, 0)),
            pl.BlockSpec((TRK, C0), lambda n, j: (n * PJ + j, 0)),
            pl.BlockSpec((2, C0), lambda n, j: (0, 0)),
        ],
        out_shape=[
            jax.ShapeDtypeStruct((NP, K), jnp.int32),
            jax.ShapeDtypeStruct((NP, C0), jnp.float32),
            jax.ShapeDtypeStruct((NP, C0), jnp.float32),
            jax.ShapeDtypeStruct((NP, C0), jnp.float32),
            jax.ShapeDtypeStruct((2, C0), jnp.float32),
        ],
        scratch_shapes=[pltpu.VMEM((2, C0), jnp.float32)],
    )(ptsp, ptsT, f2, wab, wb, wsc)


def _stats_body_nl(nl):
    def body(h_ref, cs_ref, ws_ref, stats_ref, acc):
        i = pl.program_id(0)

        @pl.when(i == 0)
        def _():
            acc[...] = jnp.zeros_like(acc)

        x = h_ref[...]
        for l in range(nl):
            x = _leaky(x * cs_ref[l, 0:1, :] + cs_ref[l, 1:2, :])
            x = jnp.dot(x, ws_ref[l], preferred_element_type=jnp.float32)
        acc[0:1, :] += jnp.sum(x, axis=0, keepdims=True)
        acc[1:2, :] += jnp.sum(x * x, axis=0, keepdims=True)
        stats_ref[...] = acc[...]
    return body


def _stats_chain(h, cs, ws):
    nl = ws.shape[0]
    rb = ROWS_B // 2
    return pl.pallas_call(
        _stats_body_nl(nl),
        grid=(NR2 // rb,),
        in_specs=[
            pl.BlockSpec((rb, 2 * C0), lambda i: (i, 0)),
            pl.BlockSpec((nl, 2, 2 * C0), lambda i: (0, 0, 0)),
            pl.BlockSpec((nl, 2 * C0, 2 * C0), lambda i: (0, 0, 0)),
        ],
        out_specs=pl.BlockSpec((2, 2 * C0), lambda i: (0, 0)),
        out_shape=jax.ShapeDtypeStruct((2, 2 * C0), jnp.float32),
        scratch_shapes=[pltpu.VMEM((2, 2 * C0), jnp.float32)],
    )(h, cs, ws)


def _final_body(h_ref, cs_ref, ws_ref, sc_ref, csc_ref, out_ref):
    x = h_ref[...]
    for l in range(2):
        x = _leaky(x * cs_ref[l, 0:1, :] + cs_ref[l, 1:2, :])
        x = jnp.dot(x, ws_ref[l], preferred_element_type=jnp.float32)
    x = _leaky(x * cs_ref[2, 0:1, :] + cs_ref[2, 1:2, :])
    pooled = jnp.mean(x.reshape(TPF, K // 2, 2 * C0), axis=1)
    pooled = 0.5 * (pooled[:, :C0] + pooled[:, C0:])
    sc = sc_ref[...] * csc_ref[0:1, :] + csc_ref[1:2, :]
    out_ref[...] = _leaky(pooled + sc)


def _final(h1, cs, ws, sc_pre, csc):
    rpb = TPF * (K // 2)
    return pl.pallas_call(
        _final_body,
        grid=(NP // TPF,),
        in_specs=[
            pl.BlockSpec((rpb, 2 * C0), lambda i: (i, 0)),
            pl.BlockSpec((3, 2, 2 * C0), lambda i: (0, 0, 0)),
            pl.BlockSpec((2, 2 * C0, 2 * C0), lambda i: (0, 0, 0)),
            pl.BlockSpec((TPF, C0), lambda i: (i, 0)),
            pl.BlockSpec((2, C0), lambda i: (0, 0)),
        ],
        out_specs=pl.BlockSpec((TPF, C0), lambda i: (i, 0)),
        out_shape=jax.ShapeDtypeStruct((NP, C0), jnp.float32),
    )(h1, cs, ws, sc_pre, csc)


NW = 32
PTS_W = NP // NW
PTS_C = 32
NCH = PTS_W // PTS_C
RPC = PTS_C * K
NR2 = NPK // 2


def _gather_body(fa_hbm, fb_hbm, idx_hbm, h1_hbm, stats_hbm,
                 idx_v0, idx_v1, fa_v0, fa_v1, g_v0, g_v1, o_v, acc_v,
                 sem_l, sem_g0, sem_g1):
    wid = lax.axis_index("s") * 2 + lax.axis_index("c")
    idx_vs, fa_vs, g_vs = (idx_v0, idx_v1), (fa_v0, fa_v1), (g_v0, g_v1)
    sem_gs = (sem_g0, sem_g1)

    def pair_fn(i, accs):
        bases = []
        loads = []
        for b in (0, 1):
            c = 2 * i + b
            base_pt = pl.multiple_of(wid * PTS_W + c * PTS_C, PTS_C)
            base_row = pl.multiple_of(base_pt * K, RPC)
            bases.append((base_pt, base_row))
            loads.append(pltpu.async_copy(
                idx_hbm.at[pl.ds(pl.multiple_of(base_row // 128, RPC // 128),
                                 RPC // 128)], idx_vs[b], sem_l))
            loads.append(pltpu.async_copy(
                fa_hbm.at[pl.ds(base_pt, PTS_C)], fa_vs[b], sem_l))
        for cp in loads:
            cp.wait()
        gathers = []
        for b in (0, 1):
            gathers.append([pltpu.async_copy(fb_hbm.at[idx_vs[b].at[j]],
                                             g_vs[b].at[pl.ds(j * 128, 128)],
                                             sem_gs[b])
                            for j in range(RPC // 128)])

        for b in (0, 1):
            for cp in gathers[b]:
                cp.wait()
            g_v = g_vs[b]
            fa_v = fa_vs[b]

            def pt_fn(p, acc2):
                a = list(acc2)
                rb = p * K
                ob = p * (K // 2)
                for kk in range(K):
                    for s in range(4):
                        h = (g_v[rb + kk, pl.ds(s * 16, 16)]
                             + fa_v[p, pl.ds(s * 16, 16)])
                        o_v[ob + kk // 2, pl.ds((kk % 2) * C0 + s * 16, 16)] = h
                        a[s] = a[s] + h
                        a[4 + s] = a[4 + s] + h * h
                return tuple(a)

            accs = lax.fori_loop(0, PTS_C, pt_fn, accs)
            base_row = bases[b][1]
            pltpu.sync_copy(o_v, h1_hbm.at[pl.ds(
                pl.multiple_of(base_row // 2, RPC // 2), RPC // 2)])
        return accs

    accs = lax.fori_loop(0, NCH // 2, pair_fn,
                         tuple([jnp.zeros((16,), jnp.float32)] * 8))
    for s in range(4):
        acc_v[0, pl.ds(s * 16, 16)] = accs[s]
        acc_v[1, pl.ds(s * 16, 16)] = accs[4 + s]
    pltpu.sync_copy(acc_v, stats_hbm.at[wid])


def _sc_gather(Fa, Fb, idx2):
    return pl.kernel(
        _gather_body,
        out_type=[
            jax.ShapeDtypeStruct((NR2, 2 * C0), jnp.float32),
            jax.ShapeDtypeStruct((NW, 2, C0), jnp.float32),
        ],
        mesh=plsc.VectorSubcoreMesh(core_axis_name="c", subcore_axis_name="s"),
        compiler_params=pltpu.CompilerParams(use_tc_tiling_on_sc=False),
        scratch_types=[
            pltpu.VMEM((RPC // 128, 128), jnp.int32),
            pltpu.VMEM((RPC // 128, 128), jnp.int32),
            pltpu.VMEM((PTS_C, C0), jnp.float32),
            pltpu.VMEM((PTS_C, C0), jnp.float32),
            pltpu.VMEM((RPC, C0), jnp.float32),
            pltpu.VMEM((RPC, C0), jnp.float32),
            pltpu.VMEM((RPC // 2, 2 * C0), jnp.float32),
            pltpu.VMEM((2, C0), jnp.float32),
            pltpu.SemaphoreType.DMA,
            pltpu.SemaphoreType.DMA,
            pltpu.SemaphoreType.DMA,
        ],
    )(Fa, Fb, idx2)


def _coef(stats, cnt, g, b):
    m = stats[0] / cnt
    v = stats[1] / cnt - m * m
    s = g * jax.lax.rsqrt(v + EPS)
    return jnp.stack([s, b - m * s])


def _dup(c):
    return jnp.concatenate([c, c], axis=1)


def _fold(st):
    return st[:, :C0] + st[:, C0:]


def _bd(w):
    z = jnp.zeros((C0, C0), w.dtype)
    return jnp.block([[w, z], [z, w]])


def kernel(points, features, W0, W1, W2, Wsc, g0, b0, g1, b1, g2, b2, gsc, bsc):
    f2 = features.reshape(NP, C0)
    W0a, W0b = W0[:C0], W0[C0:]
    ptsp = jnp.pad(points, ((0, 0), (0, 0), (0, 8 - CP)))
    ptsT = ptsp.transpose(0, 2, 1)

    idxg, Fa, Fb, sc_pre, stats_sc = _knn(ptsp, ptsT, f2, W0a - W0b, W0b, Wsc)

    h1, statsp = _sc_gather(Fa, Fb, idxg.reshape(NPK // 128, 128))
    stats1 = jnp.sum(statsp, axis=0)

    W1d, W2d = _bd(W1), _bd(W2)
    c1 = _coef(stats1, NPK, g0, b0)
    stats2 = _stats_chain(h1, _dup(c1)[None], W1d[None])
    c2 = _coef(_fold(stats2), NPK, g1, b1)
    stats3 = _stats_chain(h1, jnp.stack([_dup(c1), _dup(c2)]),
                          jnp.stack([W1d, W2d]))
    c3 = _coef(_fold(stats3), NPK, g2, b2)

    csc = _coef(stats_sc, NP, gsc, bsc)

    out = _final(h1, jnp.stack([_dup(c1), _dup(c2), _dup(c3)]),
                 jnp.stack([W1d, W2d]), sc_pre, csc)
    return out.reshape(N, P, C0)
